# baseline (speedup 1.0000x reference)
import jax
import jax.numpy as jnp
from jax.experimental import pallas as pl
from jax.experimental.pallas import tpu as pltpu

_SIZES = (8, 16, 24, 8, 8)
_OFFS = tuple(sum(_SIZES[:i]) for i in range(len(_SIZES)))
_NCHUNK = len(_SIZES)


def _pipe_kernel(x_hbm, o_hbm, in_buf, out_buf, ld_sem, st_sem):
    def load(i):
        return pltpu.make_async_copy(
            x_hbm.at[pl.ds(_OFFS[i], _SIZES[i]), :],
            in_buf.at[pl.ds(_OFFS[i], _SIZES[i]), :],
            ld_sem.at[i],
        )

    def store(i):
        return pltpu.make_async_copy(
            out_buf.at[pl.ds(_OFFS[i], _SIZES[i]), :],
            o_hbm.at[pl.ds(_OFFS[i], _SIZES[i]), :],
            st_sem.at[i],
        )

    for i in range(_NCHUNK):
        load(i).start()
    for i in range(_NCHUNK):
        load(i).wait()
        lo, hi = _OFFS[i], _OFFS[i] + _SIZES[i]
        out_buf[lo:hi, :] = jnp.tanh(jnp.tanh(in_buf[lo:hi, :]))
        store(i).start()
    for i in range(_NCHUNK):
        store(i).wait()


def kernel(x, neuron_states, enc_W, enc_b, dec_W, dec_b):
    batch, num_neurons = x.shape
    return pl.pallas_call(
        _pipe_kernel,
        in_specs=[pl.BlockSpec(memory_space=pl.ANY)],
        out_specs=pl.BlockSpec(memory_space=pl.ANY),
        out_shape=jax.ShapeDtypeStruct((batch, num_neurons), x.dtype),
        scratch_shapes=[
            pltpu.VMEM((batch, num_neurons), x.dtype),
            pltpu.VMEM((batch, num_neurons), x.dtype),
            pltpu.SemaphoreType.DMA((_NCHUNK,)),
            pltpu.SemaphoreType.DMA((_NCHUNK,)),
        ],
    )(x)


# speedup vs baseline: 1.0837x; 1.0837x over previous
import jax
import jax.numpy as jnp
from jax.experimental import pallas as pl
from jax.experimental.pallas import tpu as pltpu

_SIZES = (8, 16, 16, 16, 8)
_OFFS = tuple(sum(_SIZES[:i]) for i in range(len(_SIZES)))
_NCHUNK = len(_SIZES)


def _pipe_kernel(x_hbm, o_hbm, in_buf, out_buf, ld_sem, st_sem):
    def load(i):
        return pltpu.make_async_copy(
            x_hbm.at[pl.ds(_OFFS[i], _SIZES[i]), :],
            in_buf.at[pl.ds(_OFFS[i], _SIZES[i]), :],
            ld_sem.at[i],
        )

    def store(i):
        return pltpu.make_async_copy(
            out_buf.at[pl.ds(_OFFS[i], _SIZES[i]), :],
            o_hbm.at[pl.ds(_OFFS[i], _SIZES[i]), :],
            st_sem.at[i],
        )

    for i in range(_NCHUNK):
        load(i).start()
    for i in range(_NCHUNK):
        load(i).wait()
        lo, hi = _OFFS[i], _OFFS[i] + _SIZES[i]
        out_buf[lo:hi, :] = jnp.tanh(jnp.tanh(in_buf[lo:hi, :]))
        store(i).start()
    for i in range(_NCHUNK):
        store(i).wait()


def kernel(x, neuron_states, enc_W, enc_b, dec_W, dec_b):
    batch, num_neurons = x.shape
    return pl.pallas_call(
        _pipe_kernel,
        in_specs=[pl.BlockSpec(memory_space=pl.ANY)],
        out_specs=pl.BlockSpec(memory_space=pl.ANY),
        out_shape=jax.ShapeDtypeStruct((batch, num_neurons), x.dtype),
        scratch_shapes=[
            pltpu.VMEM((batch, num_neurons), x.dtype),
            pltpu.VMEM((batch, num_neurons), x.dtype),
            pltpu.SemaphoreType.DMA((_NCHUNK,)),
            pltpu.SemaphoreType.DMA((_NCHUNK,)),
        ],
    )(x)
